# baseline (speedup 1.0000x reference)
import jax
import jax.numpy as jnp
from jax.experimental import pallas as pl

N = 10000
E = 160000
F_IN = 33
HID = 64
HEADS = 4
NUM_LAYERS = 3
NUM_CLASSES = 8
EDGE_DIM = 4
TEMP = 2.0


def _matmul_block(x_ref, w_ref, o_ref):
    o_ref[...] = jnp.dot(x_ref[...], w_ref[...],
                         preferred_element_type=jnp.float32)


def _pallas_matmul(x, w, block_rows=1000):
    m, kdim = x.shape
    _, n = w.shape
    assert m % block_rows == 0
    grid = (m // block_rows,)
    return pl.pallas_call(
        _matmul_block,
        grid=grid,
        in_specs=[
            pl.BlockSpec((block_rows, kdim), lambda i: (i, 0)),
            pl.BlockSpec((kdim, n), lambda i: (0, 0)),
        ],
        out_specs=pl.BlockSpec((block_rows, n), lambda i: (i, 0)),
        out_shape=jax.ShapeDtypeStruct((m, n), jnp.float32),
    )(x, w)


def _layernorm(x, g, b):
    mu = jnp.mean(x, axis=-1, keepdims=True)
    var = jnp.mean((x - mu) ** 2, axis=-1, keepdims=True)
    return (x - mu) / jnp.sqrt(var + 1e-5) * g + b


def kernel(x, edge_index, edge_attr, params):
    src = edge_index[0]
    dst = edge_index[1]
    h = x @ params['node_W'] + params['node_b']
    ee = edge_attr @ params['edge_W'] + params['edge_b']
    for i in range(NUM_LAYERS):
        c = params['convs'][i]
        wqkvs = jnp.concatenate([c['Wq'], c['Wk'], c['Wv'], c['Ws']], axis=1)
        qkvs = _pallas_matmul(h, wqkvs)
        q = (qkvs[:, 0:256] + c['bq']).reshape(N, HEADS, HID)
        k = (qkvs[:, 256:512] + c['bk']).reshape(N, HEADS, HID)
        v = (qkvs[:, 512:768] + c['bv']).reshape(N, HEADS, HID)
        s = qkvs[:, 768:1024] + c['bs']
        e = (ee @ c['We']).reshape(-1, HEADS, HID)
        kj = k[src] + e
        vj = v[src] + e
        alpha = jnp.sum(q[dst] * kj, axis=-1) / jnp.sqrt(float(HID))
        amax = jax.ops.segment_max(alpha, dst, num_segments=N)
        amax = jnp.where(jnp.isfinite(amax), amax, 0.0)
        ex = jnp.exp(alpha - amax[dst])
        denom = jax.ops.segment_sum(ex, dst, num_segments=N)
        attn = ex / (denom[dst] + 1e-16)
        msg = vj * attn[..., None]
        out = jax.ops.segment_sum(msg, dst, num_segments=N).reshape(N, HEADS * HID)
        out = out + s
        hn = _pallas_matmul(out, params['proj_W']) + params['proj_b']
        hn = _layernorm(hn, params['ln_g'][i], params['ln_b'][i])
        h = h + hn
        if i < NUM_LAYERS - 1:
            h = jax.nn.relu(h)
    target = h[0:1]
    logits = (target @ params['cls_W'] + params['cls_b']) / TEMP
    return logits
